# Initial kernel scaffold; baseline (speedup 1.0000x reference)
import jax
import jax.numpy as jnp
from jax.experimental import pallas as pl


def kernel(data, table):
    raise NotImplementedError("write your pallas kernel here")


# speedup vs baseline: 1.5739x; 1.5739x over previous
import functools

import jax
import jax.numpy as jnp
from jax import lax
from jax.experimental import pallas as pl
from jax.experimental.pallas import tpu as pltpu
from jax.experimental.pallas import tpu_sc as plsc


def _gather_kernel(B, D, CH):
    info = plsc.get_sparse_core_info()
    NC, NS = info.num_cores, info.num_subcores
    NW = NC * NS
    b_per_w = B // NW
    n_chunks = b_per_w // CH
    mesh = plsc.VectorSubcoreMesh(core_axis_name="c", subcore_axis_name="s")

    @functools.partial(
        pl.kernel,
        out_type=jax.ShapeDtypeStruct((B, D), jnp.float32),
        mesh=mesh,
        scratch_types=[
            pltpu.VMEM((CH,), jnp.int32),
            pltpu.VMEM((CH, D), jnp.float32),
            pltpu.SemaphoreType.DMA,
        ],
        compiler_params=pltpu.CompilerParams(use_tc_tiling_on_sc=False),
    )
    def k(idx_hbm, table_hbm, out_hbm, idx_v, rows_v, sem):
        wid = lax.axis_index("s") * NC + lax.axis_index("c")
        base = wid * b_per_w

        @pl.loop(0, n_chunks)
        def _(i):
            off = base + i * CH
            pltpu.sync_copy(idx_hbm.at[pl.ds(off, CH)], idx_v)
            pltpu.async_copy(table_hbm.at[idx_v], rows_v, sem).wait()
            pltpu.sync_copy(rows_v, out_hbm.at[pl.ds(off, CH)])

    return k


@jax.jit
def kernel(data, table):
    B = data.size
    D = table.shape[1]
    flat = data.reshape(B)
    out = _gather_kernel(B, D, 128)(flat, table)
    return out.reshape(*data.shape, D)


# speedup vs baseline: 1.8745x; 1.1910x over previous
import functools

import jax
import jax.numpy as jnp
from jax import lax
from jax.experimental import pallas as pl
from jax.experimental.pallas import tpu as pltpu
from jax.experimental.pallas import tpu_sc as plsc


def _gather_kernel(B, D, CH, K):
    info = plsc.get_sparse_core_info()
    NC, NS = info.num_cores, info.num_subcores
    NW = NC * NS
    b_per_w = B // NW
    n_chunks = b_per_w // CH
    n_pairs = n_chunks // (2 * K)
    assert B % NW == 0 and b_per_w % CH == 0 and n_chunks % (2 * K) == 0
    mesh = plsc.VectorSubcoreMesh(core_axis_name="c", subcore_axis_name="s")

    @functools.partial(
        pl.kernel,
        out_type=jax.ShapeDtypeStruct((B, D), jnp.float32),
        mesh=mesh,
        scratch_types=[
            pltpu.VMEM((b_per_w,), jnp.int32),
            pltpu.VMEM((2 * K, CH, D), jnp.float32),
            pltpu.SemaphoreType.DMA,
            pltpu.SemaphoreType.DMA,
            pltpu.SemaphoreType.DMA,
            pltpu.SemaphoreType.DMA,
        ],
        compiler_params=pltpu.CompilerParams(use_tc_tiling_on_sc=False),
    )
    def k(idx_hbm, table_hbm, out_hbm, idx_all, rows, gsem_a, gsem_b, ssem_a, ssem_b):
        wid = lax.axis_index("s") * NC + lax.axis_index("c")
        base = wid * b_per_w
        pltpu.sync_copy(idx_hbm.at[pl.ds(base, b_per_w)], idx_all)

        def gather_desc(g, half, b, sem):
            ch = g * K + b
            idx_sl = idx_all.at[pl.ds(ch * CH, CH)]
            return pltpu.make_async_copy(
                table_hbm.at[idx_sl], rows.at[half * K + b], sem)

        def store_desc(g, half, b, sem):
            ch = g * K + b
            return pltpu.make_async_copy(
                rows.at[half * K + b], out_hbm.at[pl.ds(base + ch * CH, CH)], sem)

        def fire_gathers(g, half, sem):
            for b in range(K):
                gather_desc(g, half, b, sem).start()

        def drain_gathers(g, half, sem):
            for b in range(K):
                gather_desc(g, half, b, sem).wait()

        def fire_stores(g, half, sem):
            for b in range(K):
                store_desc(g, half, b, sem).start()

        def drain_stores(g, half, sem):
            for b in range(K):
                store_desc(g, half, b, sem).wait()

        @pl.loop(0, n_pairs)
        def _(h):
            g0 = 2 * h
            g1 = 2 * h + 1

            @pl.when(h > 0)
            def _():
                drain_stores(g0 - 2, 0, ssem_a)

            fire_gathers(g0, 0, gsem_a)
            drain_gathers(g0, 0, gsem_a)

            @pl.when(h > 0)
            def _():
                drain_stores(g1 - 2, 1, ssem_b)

            fire_gathers(g1, 1, gsem_b)
            fire_stores(g0, 0, ssem_a)
            drain_gathers(g1, 1, gsem_b)
            fire_stores(g1, 1, ssem_b)

        drain_stores(2 * n_pairs - 2, 0, ssem_a)
        drain_stores(2 * n_pairs - 1, 1, ssem_b)

    return k


@jax.jit
def kernel(data, table):
    B = data.size
    D = table.shape[1]
    flat = data.reshape(B)
    out = _gather_kernel(B, D, 128, 4)(flat, table)
    return out.reshape(*data.shape, D)
